# baseline (speedup 1.0000x reference)
import functools

import numpy as np
import jax
import jax.numpy as jnp
from jax import lax
from jax.experimental import pallas as pl
from jax.experimental.pallas import tpu as pltpu
from jax.experimental.pallas import tpu_sc as plsc

_DIM = 96
_CB = 8192
_SCALES = (1, 2, 4, 8, 16)
_B, _H, _W = 16, 16, 16
_HW = _H * _W
_RESI = 0.5
_CHUNK = 2048
_HI = lax.Precision.HIGHEST


def _interp_1d(s: int) -> np.ndarray:
    u = np.zeros((_H, s), dtype=np.float32)
    for i in range(_H):
        c = (i + 0.5) * s / _H - 0.5
        f = int(np.floor(c))
        t = c - f
        lo = min(max(f, 0), s - 1)
        hi = min(max(f + 1, 0), s - 1)
        u[i, lo] += 1.0 - t
        u[i, hi] += t
    return u


def _pool_mat(s: int) -> np.ndarray:
    k = _H // s
    p = np.zeros((s * s, _HW), dtype=np.float32)
    for ih in range(s):
        for iw in range(s):
            for dh in range(k):
                for dw in range(k):
                    p[ih * s + iw, (ih * k + dh) * _W + iw * k + dw] = 1.0 / (k * k)
    return p


_POOL = {s: _pool_mat(s) for s in _SCALES if s != _H}
_UPH = {s: np.kron(_interp_1d(s), np.eye(s, dtype=np.float32)).astype(np.float32)
        for s in _SCALES if s != _H}
_UPW = {s: np.kron(np.eye(_H, dtype=np.float32), _interp_1d(s)).astype(np.float32)
        for s in _SCALES if s != _H}


def _fused_body(n, mix, *refs):
    if mix:
        (res_ref, upp_ref, cv_ref, b_ref, p_ref, cb_ref, cbsq_ref,
         uh_ref, uw_ref, idx_ref, up_ref, res_out, loss_ref) = refs
        phi = (1.0 - _RESI) * upp_ref[0] + _RESI * (cv_ref[0] + b_ref[...])
        res = res_ref[0] - phi
        res_out[0] = res
    else:
        (res_ref, p_ref, cb_ref, cbsq_ref, uh_ref, uw_ref,
         idx_ref, up_ref, loss_ref) = refs
        res = res_ref[0]
    b = pl.program_id(0)
    pooled = lax.dot_general(p_ref[...], res, (((1,), (0,)), ((), ())),
                             preferred_element_type=jnp.float32,
                             precision=_HI)
    cols = lax.broadcasted_iota(jnp.int32, (n, _CHUNK), 1)

    def step(c, carry):
        run_min, run_idx = carry
        cb = cb_ref[pl.ds(c * _CHUNK, _CHUNK), :]
        d = cbsq_ref[:, pl.ds(c * _CHUNK, _CHUNK)] - 2.0 * lax.dot_general(
            pooled, cb, (((1,), (1,)), ((), ())),
            preferred_element_type=jnp.float32)
        mn = jnp.min(d, axis=1, keepdims=True)
        lidx = jnp.min(jnp.where(d == mn, cols, _CHUNK), axis=1,
                       keepdims=True) + c * _CHUNK
        better = mn < run_min
        return (jnp.where(better, mn, run_min),
                jnp.where(better, lidx, run_idx))

    run_min = jnp.full((n, 1), jnp.inf, jnp.float32)
    run_idx = jnp.zeros((n, 1), jnp.int32)
    _, run_idx = lax.fori_loop(0, _CB // _CHUNK, step, (run_min, run_idx))
    idx_ref[0, 0, :] = run_idx[:, 0]

    def gstep(c, q):
        cb = cb_ref[pl.ds(c * _CHUNK, _CHUNK), :]
        onehot = jnp.where(cols + c * _CHUNK == run_idx, 1.0, 0.0)
        return q + lax.dot_general(onehot, cb, (((1,), (0,)), ((), ())),
                                   preferred_element_type=jnp.float32,
                                   precision=_HI)

    q = lax.fori_loop(0, _CB // _CHUNK, gstep, jnp.zeros((n, _DIM), jnp.float32))

    diff = q - pooled
    part = jnp.sum(diff * diff).reshape(1, 1)

    @pl.when(b == 0)
    def _():
        loss_ref[...] = jnp.zeros((1, 1), jnp.float32)

    loss_ref[...] += part

    qst = pooled + (q - pooled)
    t1 = lax.dot_general(uh_ref[...], qst, (((1,), (0,)), ((), ())),
                         preferred_element_type=jnp.float32, precision=_HI)
    up_ref[0] = lax.dot_general(uw_ref[...], t1, (((1,), (0,)), ((), ())),
                                preferred_element_type=jnp.float32,
                                precision=_HI)


@functools.lru_cache(maxsize=None)
def _fused_call(s, mix):
    n = s * s
    img_spec = pl.BlockSpec((1, _HW, _DIM), lambda b: (b, 0, 0))
    in_specs = [img_spec]
    if mix:
        in_specs += [img_spec, img_spec,
                     pl.BlockSpec((1, _DIM), lambda b: (0, 0))]
    in_specs += [
        pl.BlockSpec((n, _HW), lambda b: (0, 0)),
        pl.BlockSpec((_CB, _DIM), lambda b: (0, 0)),
        pl.BlockSpec((1, _CB), lambda b: (0, 0)),
        pl.BlockSpec((_H * s, n), lambda b: (0, 0)),
        pl.BlockSpec((_HW, _H * s), lambda b: (0, 0)),
    ]
    out_specs = [
        pl.BlockSpec((1, 1, n), lambda b: (b, 0, 0)),
        img_spec,
    ]
    out_shape = [
        jax.ShapeDtypeStruct((_B, 1, n), jnp.int32),
        jax.ShapeDtypeStruct((_B, _HW, _DIM), jnp.float32),
    ]
    if mix:
        out_specs.append(img_spec)
        out_shape.append(jax.ShapeDtypeStruct((_B, _HW, _DIM), jnp.float32))
    out_specs.append(pl.BlockSpec((1, 1), lambda b: (0, 0)))
    out_shape.append(jax.ShapeDtypeStruct((1, 1), jnp.float32))
    return pl.pallas_call(
        functools.partial(_fused_body, n, mix),
        grid=(_B,),
        in_specs=in_specs,
        out_specs=out_specs,
        out_shape=out_shape,
    )


def _assign_body(n, mix, *refs):
    if mix:
        (res_ref, up_ref, cv_ref, b_ref, p_ref, cb_ref, cbsq_ref,
         idx_ref, flat_ref, res_out) = refs
        phi = (1.0 - _RESI) * up_ref[0] + _RESI * (cv_ref[0] + b_ref[...])
        res = res_ref[0] - phi
        res_out[0] = res
    else:
        res_ref, p_ref, cb_ref, cbsq_ref, idx_ref, flat_ref = refs
        res = res_ref[0]
    if n == _HW:
        pooled = res
    else:
        pooled = lax.dot_general(p_ref[...], res, (((1,), (0,)), ((), ())),
                                 preferred_element_type=jnp.float32,
                                 precision=_HI)
    flat_ref[0] = pooled

    def step(c, carry):
        run_min, run_idx = carry
        cb = cb_ref[pl.ds(c * _CHUNK, _CHUNK), :]
        d = cbsq_ref[:, pl.ds(c * _CHUNK, _CHUNK)] - 2.0 * lax.dot_general(
            pooled, cb, (((1,), (1,)), ((), ())),
            preferred_element_type=jnp.float32)
        mn = jnp.min(d, axis=1, keepdims=True)
        cols = lax.broadcasted_iota(jnp.int32, (n, _CHUNK), 1)
        lidx = jnp.min(jnp.where(d == mn, cols, _CHUNK), axis=1,
                       keepdims=True) + c * _CHUNK
        better = mn < run_min
        return (jnp.where(better, mn, run_min),
                jnp.where(better, lidx, run_idx))

    run_min = jnp.full((n, 1), jnp.inf, jnp.float32)
    run_idx = jnp.zeros((n, 1), jnp.int32)
    _, run_idx = lax.fori_loop(0, _CB // _CHUNK, step, (run_min, run_idx))
    idx_ref[0, 0, :] = run_idx[:, 0]


@functools.lru_cache(maxsize=None)
def _assign_call(s, mix):
    n = s * s
    img_spec = pl.BlockSpec((1, _HW, _DIM), lambda b: (b, 0, 0))
    in_specs = [img_spec]
    if mix:
        in_specs += [img_spec, img_spec,
                     pl.BlockSpec((1, _DIM), lambda b: (0, 0))]
    in_specs += [
        pl.BlockSpec((n, _HW), lambda b: (0, 0)),
        pl.BlockSpec((_CB, _DIM), lambda b: (0, 0)),
        pl.BlockSpec((1, _CB), lambda b: (0, 0)),
    ]
    out_specs = [
        pl.BlockSpec((1, 1, n), lambda b: (b, 0, 0)),
        pl.BlockSpec((1, n, _DIM), lambda b: (b, 0, 0)),
    ]
    out_shape = [
        jax.ShapeDtypeStruct((_B, 1, n), jnp.int32),
        jax.ShapeDtypeStruct((_B, n, _DIM), jnp.float32),
    ]
    if mix:
        out_specs.append(img_spec)
        out_shape.append(jax.ShapeDtypeStruct((_B, _HW, _DIM), jnp.float32))
    return pl.pallas_call(
        functools.partial(_assign_body, n, mix),
        grid=(_B,),
        in_specs=in_specs,
        out_specs=out_specs,
        out_shape=out_shape,
    )


_GDIM = 128


@functools.lru_cache(maxsize=None)
def _gather_call(nrows):
    info = plsc.get_sparse_core_info()
    nw = info.num_cores * info.num_subcores
    per_w = nrows // nw
    mesh = plsc.VectorSubcoreMesh(core_axis_name="c", subcore_axis_name="s")

    @functools.partial(
        pl.kernel,
        mesh=mesh,
        out_type=jax.ShapeDtypeStruct((nrows, _GDIM), jnp.float32),
        scratch_types=[
            pltpu.VMEM((per_w,), jnp.int32),
            pltpu.VMEM((per_w, _GDIM), jnp.float32),
            pltpu.SemaphoreType.DMA,
        ],
    )
    def k(table_hbm, idx_hbm, out_hbm, idx_v, rows_v, sem):
        wid = lax.axis_index("s") * info.num_cores + lax.axis_index("c")
        base = wid * per_w
        pltpu.sync_copy(idx_hbm.at[pl.ds(base, per_w)], idx_v)
        pltpu.async_copy(table_hbm.at[idx_v], rows_v, sem).wait()
        pltpu.sync_copy(rows_v, out_hbm.at[pl.ds(base, per_w)])

    return k


def _gather_rows(codebook_padded, idx_flat):
    nrows = idx_flat.shape[0]
    npad = max(256, nrows)
    if nrows < npad:
        idx_flat = jnp.concatenate(
            [idx_flat, jnp.zeros((npad - nrows,), jnp.int32)])
    return _gather_call(npad)(codebook_padded, idx_flat)[:nrows, :_DIM]


def _up_body(s, *refs):
    if s == _H:
        q_ref, flat_ref, up_ref, loss_ref = refs
    else:
        q_ref, flat_ref, uh_ref, uw_ref, up_ref, loss_ref = refs
    b = pl.program_id(0)
    q = q_ref[0]
    flat = flat_ref[0]
    diff = q - flat
    part = jnp.sum(diff * diff).reshape(1, 1)

    @pl.when(b == 0)
    def _():
        loss_ref[...] = jnp.zeros((1, 1), jnp.float32)

    loss_ref[...] += part

    qst = flat + (q - flat)
    if s == _H:
        up_ref[0] = qst
    else:
        t1 = lax.dot_general(uh_ref[...], qst, (((1,), (0,)), ((), ())),
                             preferred_element_type=jnp.float32,
                             precision=_HI)
        up_ref[0] = lax.dot_general(uw_ref[...], t1, (((1,), (0,)), ((), ())),
                                    preferred_element_type=jnp.float32,
                                    precision=_HI)


@functools.lru_cache(maxsize=None)
def _up_call(s):
    n = s * s
    img_spec = pl.BlockSpec((1, _HW, _DIM), lambda b: (b, 0, 0))
    in_specs = [
        pl.BlockSpec((1, n, _DIM), lambda b: (b, 0, 0)),
        pl.BlockSpec((1, n, _DIM), lambda b: (b, 0, 0)),
    ]
    if s != _H:
        in_specs.append(pl.BlockSpec((_H * s, n), lambda b: (0, 0)))
        in_specs.append(pl.BlockSpec((_HW, _H * s), lambda b: (0, 0)))
    return pl.pallas_call(
        functools.partial(_up_body, s),
        grid=(_B,),
        in_specs=in_specs,
        out_specs=[img_spec, pl.BlockSpec((1, 1), lambda b: (0, 0))],
        out_shape=[jax.ShapeDtypeStruct((_B, _HW, _DIM), jnp.float32),
                   jax.ShapeDtypeStruct((1, 1), jnp.float32)],
    )


def _mix_body(last, *refs):
    if last:
        up_ref, cv_ref, res_ref, b_ref, x_ref, res_out, recon_out = refs
    else:
        up_ref, cv_ref, res_ref, b_ref, res_out = refs
    phi = (1.0 - _RESI) * up_ref[0] + _RESI * (cv_ref[0] + b_ref[...])
    new_res = res_ref[0] - phi
    res_out[0] = new_res
    if last:
        recon_out[0] = x_ref[0] - new_res


@functools.lru_cache(maxsize=None)
def _mix_call(last):
    img_spec = pl.BlockSpec((1, _HW, _DIM), lambda b: (b, 0, 0))
    in_specs = [img_spec, img_spec, img_spec,
                pl.BlockSpec((1, _DIM), lambda b: (0, 0))]
    out_specs = [img_spec]
    out_shape = [jax.ShapeDtypeStruct((_B, _HW, _DIM), jnp.float32)]
    if last:
        in_specs.append(img_spec)
        out_specs.append(img_spec)
        out_shape.append(jax.ShapeDtypeStruct((_B, _HW, _DIM), jnp.float32))
    return pl.pallas_call(
        functools.partial(_mix_body, last),
        grid=(_B,),
        in_specs=in_specs,
        out_specs=out_specs,
        out_shape=out_shape,
    )


def kernel(x, codebook, conv_w, conv_b):
    xt = x.transpose(0, 2, 3, 1).reshape(_B, _HW, _DIM)
    cb_padded = jnp.pad(codebook, ((0, 0), (0, _GDIM - _DIM)))
    cbsq_row = jnp.sum(codebook ** 2, axis=1).reshape(1, _CB)
    bias = conv_b.reshape(1, _DIM)
    w_hwio = conv_w.transpose(2, 3, 1, 0)
    residual = xt
    up = cvt = None
    idx_list = []
    loss = jnp.float32(0.0)
    for si, s in enumerate(_SCALES):
        n = s * s
        if s != _H:
            args = [residual]
            if si > 0:
                args += [up, cvt, bias]
            args += [jnp.asarray(_POOL[s]), codebook, cbsq_row,
                     jnp.asarray(_UPH[s]), jnp.asarray(_UPW[s])]
            outs = _fused_call(s, si > 0)(*args)
            if si > 0:
                idx3, up, residual, lpart = outs
            else:
                idx3, up, lpart = outs
        else:
            idx3, flat, residual = _assign_call(s, True)(
                residual, up, cvt, bias, jnp.zeros((n, _HW), jnp.float32),
                codebook, cbsq_row)
            q = _gather_rows(cb_padded, idx3.reshape(_B * n)).reshape(
                _B, n, _DIM)
            up, lpart = _up_call(s)(q, flat)
        cv = lax.conv_general_dilated(
            up.reshape(_B, _H, _W, _DIM), w_hwio, window_strides=(1, 1),
            padding='SAME', dimension_numbers=('NHWC', 'HWIO', 'NHWC'))
        cvt = cv.reshape(_B, _HW, _DIM)
        loss = loss + lpart[0, 0] / (_B * n * _DIM)
        idx_list.append(idx3.reshape(_B, s, s))
    residual, recon = _mix_call(True)(up, cvt, residual, bias, xt)
    recon_out = recon.reshape(_B, _H, _W, _DIM).transpose(0, 3, 1, 2)
    return (recon_out, tuple(idx_list), loss / len(_SCALES))


# speedup vs baseline: 1.2222x; 1.2222x over previous
import functools

import numpy as np
import jax
import jax.numpy as jnp
from jax import lax
from jax.experimental import pallas as pl
from jax.experimental.pallas import tpu as pltpu
from jax.experimental.pallas import tpu_sc as plsc

_DIM = 96
_CB = 8192
_SCALES = (1, 2, 4, 8, 16)
_B, _H, _W = 16, 16, 16
_HW = _H * _W
_RESI = 0.5
_CHUNK = 2048
_HI = lax.Precision.HIGHEST


def _interp_1d(s: int) -> np.ndarray:
    u = np.zeros((_H, s), dtype=np.float32)
    for i in range(_H):
        c = (i + 0.5) * s / _H - 0.5
        f = int(np.floor(c))
        t = c - f
        lo = min(max(f, 0), s - 1)
        hi = min(max(f + 1, 0), s - 1)
        u[i, lo] += 1.0 - t
        u[i, hi] += t
    return u


def _pool_mat(s: int) -> np.ndarray:
    k = _H // s
    p = np.zeros((s * s, _HW), dtype=np.float32)
    for ih in range(s):
        for iw in range(s):
            for dh in range(k):
                for dw in range(k):
                    p[ih * s + iw, (ih * k + dh) * _W + iw * k + dw] = 1.0 / (k * k)
    return p


_POOL = {s: _pool_mat(s) for s in _SCALES if s != _H}
_UPH = {s: np.kron(_interp_1d(s), np.eye(s, dtype=np.float32)).astype(np.float32)
        for s in _SCALES if s != _H}
_UPW = {s: np.kron(np.eye(_H, dtype=np.float32), _interp_1d(s)).astype(np.float32)
        for s in _SCALES if s != _H}


def _fused_body(n, mix, *refs):
    if mix:
        (res_ref, upp_ref, cv_ref, b_ref, p_ref, cb_ref, cbsq_ref,
         uh_ref, uw_ref, idx_ref, up_ref, res_out, loss_ref) = refs
        phi = (1.0 - _RESI) * upp_ref[0] + _RESI * (cv_ref[0] + b_ref[...])
        res = res_ref[0] - phi
        res_out[0] = res
    else:
        (res_ref, p_ref, cb_ref, cbsq_ref, uh_ref, uw_ref,
         idx_ref, up_ref, loss_ref) = refs
        res = res_ref[0]
    b = pl.program_id(0)
    pooled = lax.dot_general(p_ref[...], res, (((1,), (0,)), ((), ())),
                             preferred_element_type=jnp.float32,
                             precision=_HI)
    cols = lax.broadcasted_iota(jnp.int32, (n, _CHUNK), 1)

    def step(c, carry):
        run_min, run_idx = carry
        cb = cb_ref[pl.ds(c * _CHUNK, _CHUNK), :]
        d = cbsq_ref[:, pl.ds(c * _CHUNK, _CHUNK)] - 2.0 * lax.dot_general(
            pooled, cb, (((1,), (1,)), ((), ())),
            preferred_element_type=jnp.float32)
        mn = jnp.min(d, axis=1, keepdims=True)
        lidx = jnp.min(jnp.where(d == mn, cols, _CHUNK), axis=1,
                       keepdims=True) + c * _CHUNK
        better = mn < run_min
        return (jnp.where(better, mn, run_min),
                jnp.where(better, lidx, run_idx))

    run_min = jnp.full((n, 1), jnp.inf, jnp.float32)
    run_idx = jnp.zeros((n, 1), jnp.int32)
    _, run_idx = lax.fori_loop(0, _CB // _CHUNK, step, (run_min, run_idx))
    idx_ref[0, 0, :] = run_idx[:, 0]

    def gstep(c, q):
        cb = cb_ref[pl.ds(c * _CHUNK, _CHUNK), :]
        onehot = jnp.where(cols + c * _CHUNK == run_idx, 1.0, 0.0)
        return q + lax.dot_general(onehot, cb, (((1,), (0,)), ((), ())),
                                   preferred_element_type=jnp.float32,
                                   precision=_HI)

    q = lax.fori_loop(0, _CB // _CHUNK, gstep, jnp.zeros((n, _DIM), jnp.float32))

    diff = q - pooled
    part = jnp.sum(diff * diff).reshape(1, 1)

    @pl.when(b == 0)
    def _():
        loss_ref[...] = jnp.zeros((1, 1), jnp.float32)

    loss_ref[...] += part

    qst = pooled + (q - pooled)
    t1 = lax.dot_general(uh_ref[...], qst, (((1,), (0,)), ((), ())),
                         preferred_element_type=jnp.float32, precision=_HI)
    up_ref[0] = lax.dot_general(uw_ref[...], t1, (((1,), (0,)), ((), ())),
                                preferred_element_type=jnp.float32,
                                precision=_HI)


@functools.lru_cache(maxsize=None)
def _fused_call(s, mix):
    n = s * s
    img_spec = pl.BlockSpec((1, _HW, _DIM), lambda b: (b, 0, 0))
    in_specs = [img_spec]
    if mix:
        in_specs += [img_spec, img_spec,
                     pl.BlockSpec((1, _DIM), lambda b: (0, 0))]
    in_specs += [
        pl.BlockSpec((n, _HW), lambda b: (0, 0)),
        pl.BlockSpec((_CB, _DIM), lambda b: (0, 0)),
        pl.BlockSpec((1, _CB), lambda b: (0, 0)),
        pl.BlockSpec((_H * s, n), lambda b: (0, 0)),
        pl.BlockSpec((_HW, _H * s), lambda b: (0, 0)),
    ]
    out_specs = [
        pl.BlockSpec((1, 1, n), lambda b: (b, 0, 0)),
        img_spec,
    ]
    out_shape = [
        jax.ShapeDtypeStruct((_B, 1, n), jnp.int32),
        jax.ShapeDtypeStruct((_B, _HW, _DIM), jnp.float32),
    ]
    if mix:
        out_specs.append(img_spec)
        out_shape.append(jax.ShapeDtypeStruct((_B, _HW, _DIM), jnp.float32))
    out_specs.append(pl.BlockSpec((1, 1), lambda b: (0, 0)))
    out_shape.append(jax.ShapeDtypeStruct((1, 1), jnp.float32))
    return pl.pallas_call(
        functools.partial(_fused_body, n, mix),
        grid=(_B,),
        in_specs=in_specs,
        out_specs=out_specs,
        out_shape=out_shape,
    )


def _assign_body(n, mix, *refs):
    if mix:
        (res_ref, up_ref, cv_ref, b_ref, p_ref, cb_ref, cbsq_ref,
         idx_ref, flat_ref, res_out) = refs
        phi = (1.0 - _RESI) * up_ref[0] + _RESI * (cv_ref[0] + b_ref[...])
        res = res_ref[0] - phi
        res_out[0] = res
    else:
        res_ref, p_ref, cb_ref, cbsq_ref, idx_ref, flat_ref = refs
        res = res_ref[0]
    if n == _HW:
        pooled = res
    else:
        pooled = lax.dot_general(p_ref[...], res, (((1,), (0,)), ((), ())),
                                 preferred_element_type=jnp.float32,
                                 precision=_HI)
    flat_ref[0] = pooled

    def step(c, carry):
        run_min, run_idx = carry
        cb = cb_ref[pl.ds(c * _CHUNK, _CHUNK), :]
        d = cbsq_ref[:, pl.ds(c * _CHUNK, _CHUNK)] - 2.0 * lax.dot_general(
            pooled, cb, (((1,), (1,)), ((), ())),
            preferred_element_type=jnp.float32)
        mn = jnp.min(d, axis=1, keepdims=True)
        cols = lax.broadcasted_iota(jnp.int32, (n, _CHUNK), 1)
        lidx = jnp.min(jnp.where(d == mn, cols, _CHUNK), axis=1,
                       keepdims=True) + c * _CHUNK
        better = mn < run_min
        return (jnp.where(better, mn, run_min),
                jnp.where(better, lidx, run_idx))

    run_min = jnp.full((n, 1), jnp.inf, jnp.float32)
    run_idx = jnp.zeros((n, 1), jnp.int32)
    _, run_idx = lax.fori_loop(0, _CB // _CHUNK, step, (run_min, run_idx))
    idx_ref[0, 0, :] = run_idx[:, 0]


@functools.lru_cache(maxsize=None)
def _assign_call(s, mix):
    n = s * s
    img_spec = pl.BlockSpec((1, _HW, _DIM), lambda b: (b, 0, 0))
    in_specs = [img_spec]
    if mix:
        in_specs += [img_spec, img_spec,
                     pl.BlockSpec((1, _DIM), lambda b: (0, 0))]
    in_specs += [
        pl.BlockSpec((n, _HW), lambda b: (0, 0)),
        pl.BlockSpec((_CB, _DIM), lambda b: (0, 0)),
        pl.BlockSpec((1, _CB), lambda b: (0, 0)),
    ]
    out_specs = [
        pl.BlockSpec((1, 1, n), lambda b: (b, 0, 0)),
        pl.BlockSpec((1, n, _DIM), lambda b: (b, 0, 0)),
    ]
    out_shape = [
        jax.ShapeDtypeStruct((_B, 1, n), jnp.int32),
        jax.ShapeDtypeStruct((_B, n, _DIM), jnp.float32),
    ]
    if mix:
        out_specs.append(img_spec)
        out_shape.append(jax.ShapeDtypeStruct((_B, _HW, _DIM), jnp.float32))
    return pl.pallas_call(
        functools.partial(_assign_body, n, mix),
        grid=(_B,),
        in_specs=in_specs,
        out_specs=out_specs,
        out_shape=out_shape,
    )


_GDIM = 128


@functools.lru_cache(maxsize=None)
def _gather_call(nrows):
    info = plsc.get_sparse_core_info()
    nw = info.num_cores * info.num_subcores
    per_w = nrows // nw
    mesh = plsc.VectorSubcoreMesh(core_axis_name="c", subcore_axis_name="s")

    @functools.partial(
        pl.kernel,
        mesh=mesh,
        out_type=jax.ShapeDtypeStruct((nrows, _GDIM), jnp.float32),
        scratch_types=[
            pltpu.VMEM((per_w,), jnp.int32),
            pltpu.VMEM((per_w, _GDIM), jnp.float32),
            pltpu.SemaphoreType.DMA,
        ],
    )
    def k(table_hbm, idx_hbm, out_hbm, idx_v, rows_v, sem):
        wid = lax.axis_index("s") * info.num_cores + lax.axis_index("c")
        base = wid * per_w
        pltpu.sync_copy(idx_hbm.at[pl.ds(base, per_w)], idx_v)
        pltpu.async_copy(table_hbm.at[idx_v], rows_v, sem).wait()
        pltpu.sync_copy(rows_v, out_hbm.at[pl.ds(base, per_w)])

    return k


def _gather_rows(codebook_padded, idx_flat):
    nrows = idx_flat.shape[0]
    npad = max(256, nrows)
    if nrows < npad:
        idx_flat = jnp.concatenate(
            [idx_flat, jnp.zeros((npad - nrows,), jnp.int32)])
    return _gather_call(npad)(codebook_padded, idx_flat)[:nrows, :_DIM]


def _up_body(s, *refs):
    if s == _H:
        q_ref, flat_ref, up_ref, loss_ref = refs
    else:
        q_ref, flat_ref, uh_ref, uw_ref, up_ref, loss_ref = refs
    b = pl.program_id(0)
    q = q_ref[0]
    flat = flat_ref[0]
    diff = q - flat
    part = jnp.sum(diff * diff).reshape(1, 1)

    @pl.when(b == 0)
    def _():
        loss_ref[...] = jnp.zeros((1, 1), jnp.float32)

    loss_ref[...] += part

    qst = flat + (q - flat)
    if s == _H:
        up_ref[0] = qst
    else:
        t1 = lax.dot_general(uh_ref[...], qst, (((1,), (0,)), ((), ())),
                             preferred_element_type=jnp.float32,
                             precision=_HI)
        up_ref[0] = lax.dot_general(uw_ref[...], t1, (((1,), (0,)), ((), ())),
                                    preferred_element_type=jnp.float32,
                                    precision=_HI)


@functools.lru_cache(maxsize=None)
def _up_call(s):
    n = s * s
    img_spec = pl.BlockSpec((1, _HW, _DIM), lambda b: (b, 0, 0))
    in_specs = [
        pl.BlockSpec((1, n, _DIM), lambda b: (b, 0, 0)),
        pl.BlockSpec((1, n, _DIM), lambda b: (b, 0, 0)),
    ]
    if s != _H:
        in_specs.append(pl.BlockSpec((_H * s, n), lambda b: (0, 0)))
        in_specs.append(pl.BlockSpec((_HW, _H * s), lambda b: (0, 0)))
    return pl.pallas_call(
        functools.partial(_up_body, s),
        grid=(_B,),
        in_specs=in_specs,
        out_specs=[img_spec, pl.BlockSpec((1, 1), lambda b: (0, 0))],
        out_shape=[jax.ShapeDtypeStruct((_B, _HW, _DIM), jnp.float32),
                   jax.ShapeDtypeStruct((1, 1), jnp.float32)],
    )


def _mix_body(last, *refs):
    if last:
        up_ref, cv_ref, res_ref, b_ref, x_ref, res_out, recon_out = refs
    else:
        up_ref, cv_ref, res_ref, b_ref, res_out = refs
    phi = (1.0 - _RESI) * up_ref[0] + _RESI * (cv_ref[0] + b_ref[...])
    new_res = res_ref[0] - phi
    res_out[0] = new_res
    if last:
        recon_out[0] = x_ref[0] - new_res


@functools.lru_cache(maxsize=None)
def _mix_call(last):
    img_spec = pl.BlockSpec((1, _HW, _DIM), lambda b: (b, 0, 0))
    in_specs = [img_spec, img_spec, img_spec,
                pl.BlockSpec((1, _DIM), lambda b: (0, 0))]
    out_specs = [img_spec]
    out_shape = [jax.ShapeDtypeStruct((_B, _HW, _DIM), jnp.float32)]
    if last:
        in_specs.append(img_spec)
        out_specs.append(img_spec)
        out_shape.append(jax.ShapeDtypeStruct((_B, _HW, _DIM), jnp.float32))
    return pl.pallas_call(
        functools.partial(_mix_body, last),
        grid=(_B,),
        in_specs=in_specs,
        out_specs=out_specs,
        out_shape=out_shape,
    )


def kernel(x, codebook, conv_w, conv_b):
    xt = x.transpose(0, 2, 3, 1).reshape(_B, _HW, _DIM)
    cb_padded = jnp.pad(codebook, ((0, 0), (0, _GDIM - _DIM)))
    cbsq_row = jnp.sum(codebook ** 2, axis=1).reshape(1, _CB)
    bias = conv_b.reshape(1, _DIM)
    w_hwio = conv_w.transpose(2, 3, 1, 0)
    residual = xt
    up = cvt = None
    idx_list = []
    loss = jnp.float32(0.0)
    for si, s in enumerate(_SCALES):
        n = s * s
        pmat = jnp.zeros((n, _HW), jnp.float32) if s == _H \
            else jnp.asarray(_POOL[s])
        if si == 0:
            idx3, flat = _assign_call(s, False)(
                residual, pmat, codebook, cbsq_row)
        else:
            idx3, flat, residual = _assign_call(s, True)(
                residual, up, cvt, bias, pmat, codebook, cbsq_row)
        q = _gather_rows(cb_padded, idx3.reshape(_B * n)).reshape(_B, n, _DIM)
        args = [q, flat]
        if s != _H:
            args.append(jnp.asarray(_UPH[s]))
            args.append(jnp.asarray(_UPW[s]))
        up, lpart = _up_call(s)(*args)
        cv = lax.conv_general_dilated(
            up.reshape(_B, _H, _W, _DIM), w_hwio, window_strides=(1, 1),
            padding='SAME', dimension_numbers=('NHWC', 'HWIO', 'NHWC'))
        cvt = cv.reshape(_B, _HW, _DIM)
        loss = loss + lpart[0, 0] / (_B * n * _DIM)
        idx_list.append(idx3.reshape(_B, s, s))
    residual, recon = _mix_call(True)(up, cvt, residual, bias, xt)
    recon_out = recon.reshape(_B, _H, _W, _DIM).transpose(0, 3, 1, 2)
    return (recon_out, tuple(idx_list), loss / len(_SCALES))
